# baseline (speedup 1.0000x reference)
import functools

import jax
import jax.numpy as jnp
from jax import lax
from jax.experimental import pallas as pl
from jax.experimental.pallas import tpu as pltpu
from jax.experimental.pallas import tpu_sc as plsc


def _argmin_body(nk, ka, x_ref, e_ref, ind_ref, acc_ref, xa_ref, ea_ref):
    ti = pl.program_id(0)
    kj = pl.program_id(1)
    tm = x_ref.shape[0]
    dim = x_ref.shape[1]

    @pl.when(kj == 0)
    def _():
        xa_ref[:, pl.ds(0, dim)] = x_ref[...].astype(jnp.bfloat16)
        colx = lax.broadcasted_iota(jnp.int32, (tm, ka - dim), 1)
        xa_ref[:, pl.ds(dim, ka - dim)] = jnp.where(
            colx == 0, 1.0, 0.0).astype(jnp.bfloat16)

    @pl.when(ti == 0)
    def _():
        es = e_ref[pl.ds(kj * nk, nk), :]
        ea_ref[pl.ds(kj * nk, nk), pl.ds(0, dim)] = es.astype(jnp.bfloat16)
        half = jnp.full((1, dim), 0.5, dtype=jnp.float32)
        hencol = lax.dot_general(es * es, half, (((1,), (1,)), ((), ())),
                                 preferred_element_type=jnp.float32)
        cole = lax.broadcasted_iota(jnp.int32, (nk, ka - dim), 1)
        ea_ref[pl.ds(kj * nk, nk), pl.ds(dim, ka - dim)] = jnp.where(
            cole == 0, -hencol, 0.0).astype(jnp.bfloat16)

    t = lax.dot_general(xa_ref[...], ea_ref[pl.ds(kj * nk, nk), :],
                        (((1,), (1,)), ((), ())),
                        preferred_element_type=jnp.float32)
    lane = lax.broadcasted_iota(jnp.int32, (1, nk), 1)
    jbits = lane | (kj * nk)
    tp = lax.bitcast_convert_type(
        (lax.bitcast_convert_type(t, jnp.int32) & jnp.int32(~8191)) | jbits,
        jnp.float32)
    bm = jnp.max(tp, axis=1, keepdims=True)

    @pl.when(kj == 0)
    def _():
        acc_ref[...] = bm

    @pl.when(kj > 0)
    def _():
        acc_ref[...] = jnp.maximum(acc_ref[...], bm)

    @pl.when(kj == pl.num_programs(1) - 1)
    def _():
        ind_ref[...] = lax.bitcast_convert_type(
            acc_ref[...], jnp.int32) & jnp.int32(8191)


def _argmin_call(x, e, tm, nk, ka):
    b, dim = x.shape
    n_embed = e.shape[0]
    return pl.pallas_call(
        functools.partial(_argmin_body, nk, ka),
        grid=(b // tm, n_embed // nk),
        in_specs=[
            pl.BlockSpec((tm, dim), lambda i, j: (i, 0)),
            pl.BlockSpec((n_embed, dim), lambda i, j: (0, 0)),
        ],
        out_specs=pl.BlockSpec((tm, 1), lambda i, j: (i, 0)),
        out_shape=jax.ShapeDtypeStruct((b, 1), jnp.int32),
        scratch_shapes=[
            pltpu.VMEM((tm, 1), jnp.float32),
            pltpu.VMEM((tm, ka), jnp.bfloat16),
            pltpu.VMEM((n_embed, ka), jnp.bfloat16),
        ],
    )(x, e)


def _make_sc_gather(dim, b):
    info = plsc.get_sparse_core_info()
    nc, ns = info.num_cores, info.num_subcores
    nw = nc * ns
    b_per_w = b // nw
    ch = 128
    n_chunks = b_per_w // ch
    mesh = plsc.VectorSubcoreMesh(core_axis_name="c", subcore_axis_name="s")

    @functools.partial(
        pl.kernel, mesh=mesh,
        out_type=jax.ShapeDtypeStruct((b, dim), jnp.float32),
        scratch_types=[
            pltpu.VMEM((b_per_w,), jnp.int32),
            pltpu.VMEM((ch, dim), jnp.float32),
            pltpu.VMEM((ch, dim), jnp.float32),
            pltpu.SemaphoreType.DMA,
            pltpu.SemaphoreType.DMA,
            pltpu.SemaphoreType.DMA,
            pltpu.SemaphoreType.DMA,
        ],
    )
    def gather_kernel(table_hbm, idx_hbm, out_hbm, idx_v, buf0, buf1,
                      g0, g1, w0, w1):
        wid = lax.axis_index("s") * nc + lax.axis_index("c")
        base = wid * b_per_w
        pltpu.sync_copy(idx_hbm.at[pl.ds(base, b_per_w)], idx_v)
        bufs = (buf0, buf1)
        gsem = (g0, g1)
        wsem = (w0, w1)
        gcp = [pltpu.async_copy(table_hbm.at[idx_v.at[pl.ds(c * ch, ch)]],
                                bufs[c], gsem[c])
               for c in range(min(2, n_chunks))]
        wcp = [None, None]
        for c in range(n_chunks):
            s = c & 1
            gcp[s].wait()
            wcp[s] = pltpu.async_copy(
                bufs[s], out_hbm.at[pl.ds(base + c * ch, ch)], wsem[s])
            if c + 2 < n_chunks:
                wcp[s].wait()
                gcp[s] = pltpu.async_copy(
                    table_hbm.at[idx_v.at[pl.ds((c + 2) * ch, ch)]],
                    bufs[s], gsem[s])
        wcp[(n_chunks - 2) & 1].wait()
        wcp[(n_chunks - 1) & 1].wait()

    return gather_kernel


def _proj_body(x_ref, q_ref, w_ref, b_ref, out_ref):
    x = x_ref[...]
    r = x - q_ref[...]
    out_ref[...] = (x + b_ref[...]
                    + lax.dot_general(r, w_ref[...], (((1,), (1,)), ((), ())),
                                      preferred_element_type=jnp.float32))


def _proj_call(x, q, w, bias, tm):
    b, dim = x.shape
    dim_out = w.shape[0]
    return pl.pallas_call(
        _proj_body,
        grid=(b // tm,),
        in_specs=[
            pl.BlockSpec((tm, dim), lambda i: (i, 0)),
            pl.BlockSpec((tm, dim), lambda i: (i, 0)),
            pl.BlockSpec((dim_out, dim), lambda i: (0, 0)),
            pl.BlockSpec((1, dim_out), lambda i: (0, 0)),
        ],
        out_specs=pl.BlockSpec((tm, dim_out), lambda i: (i, 0)),
        out_shape=jax.ShapeDtypeStruct((b, dim_out), jnp.float32),
    )(x, q, w, bias.reshape(1, dim_out))


def kernel(x, embed_weight, proj_w, proj_b):
    b, dim = x.shape
    ind = _argmin_call(x, embed_weight, tm=4096, nk=512, ka=264).reshape(b)
    quantized = _make_sc_gather(dim, b)(embed_weight, ind)
    return _proj_call(x, quantized, proj_w, proj_b, tm=1024)


# speedup vs baseline: 1.4815x; 1.4815x over previous
import functools

import jax
import jax.numpy as jnp
from jax import lax
from jax.experimental import pallas as pl
from jax.experimental.pallas import tpu as pltpu
from jax.experimental.pallas import tpu_sc as plsc


def _argmin_body(nk, x_ref, e_ref, ind_ref, acc_ref, hen_ref):
    ti = pl.program_id(0)
    kj = pl.program_id(1)
    dim = x_ref.shape[1]
    e = e_ref[pl.ds(kj * nk, nk), :].astype(jnp.bfloat16)

    @pl.when(ti == 0)
    def _():
        half = jnp.full((1, dim), 0.5, dtype=jnp.bfloat16)
        hen_ref[:, pl.ds(kj * nk, nk)] = lax.dot_general(
            half, e * e, (((1,), (1,)), ((), ())),
            preferred_element_type=jnp.float32)

    dot = lax.dot_general(x_ref[...].astype(jnp.bfloat16), e,
                          (((1,), (1,)), ((), ())),
                          preferred_element_type=jnp.float32)
    t = dot - hen_ref[:, pl.ds(kj * nk, nk)]
    lane = lax.broadcasted_iota(jnp.int32, (1, nk), 1)
    jbits = lane | (kj * nk)
    tp = lax.bitcast_convert_type(
        (lax.bitcast_convert_type(t, jnp.int32) & jnp.int32(~8191)) | jbits,
        jnp.float32)
    bm = jnp.max(tp, axis=1, keepdims=True)

    @pl.when(kj == 0)
    def _():
        acc_ref[...] = bm

    @pl.when(kj > 0)
    def _():
        acc_ref[...] = jnp.maximum(acc_ref[...], bm)

    @pl.when(kj == pl.num_programs(1) - 1)
    def _():
        ind_ref[...] = lax.bitcast_convert_type(
            acc_ref[...], jnp.int32) & jnp.int32(8191)


def _argmin_call(x, e, tm, nk):
    b, dim = x.shape
    n_embed = e.shape[0]
    return pl.pallas_call(
        functools.partial(_argmin_body, nk),
        grid=(b // tm, n_embed // nk),
        in_specs=[
            pl.BlockSpec((tm, dim), lambda i, j: (i, 0)),
            pl.BlockSpec((n_embed, dim), lambda i, j: (0, 0)),
        ],
        out_specs=pl.BlockSpec((tm, 1), lambda i, j: (i, 0)),
        out_shape=jax.ShapeDtypeStruct((b, 1), jnp.int32),
        scratch_shapes=[
            pltpu.VMEM((tm, 1), jnp.float32),
            pltpu.VMEM((1, n_embed), jnp.float32),
        ],
    )(x, e)


def _make_sc_gather(dim, b):
    info = plsc.get_sparse_core_info()
    nc, ns = info.num_cores, info.num_subcores
    nw = nc * ns
    b_per_w = b // nw
    ch = 128
    n_chunks = b_per_w // ch
    mesh = plsc.VectorSubcoreMesh(core_axis_name="c", subcore_axis_name="s")

    @functools.partial(
        pl.kernel, mesh=mesh,
        out_type=jax.ShapeDtypeStruct((b, dim), jnp.float32),
        scratch_types=[
            pltpu.VMEM((b_per_w,), jnp.int32),
            pltpu.VMEM((ch, dim), jnp.float32),
            pltpu.SemaphoreType.DMA,
        ],
    )
    def gather_kernel(table_hbm, idx_hbm, out_hbm, idx_v, buf, sem):
        wid = lax.axis_index("s") * nc + lax.axis_index("c")
        base = wid * b_per_w
        pltpu.sync_copy(idx_hbm.at[pl.ds(base, b_per_w)], idx_v)
        for c in range(n_chunks):
            pltpu.async_copy(
                table_hbm.at[idx_v.at[pl.ds(c * ch, ch)]], buf, sem).wait()
            pltpu.sync_copy(buf, out_hbm.at[pl.ds(base + c * ch, ch)])

    return gather_kernel


def _proj_body(x_ref, q_ref, w_ref, b_ref, out_ref):
    x = x_ref[...]
    r = x - q_ref[...]
    out_ref[...] = (x + b_ref[...]
                    + lax.dot_general(r, w_ref[...], (((1,), (1,)), ((), ())),
                                      preferred_element_type=jnp.float32))


def _proj_call(x, q, w, bias, tm):
    b, dim = x.shape
    dim_out = w.shape[0]
    return pl.pallas_call(
        _proj_body,
        grid=(b // tm,),
        in_specs=[
            pl.BlockSpec((tm, dim), lambda i: (i, 0)),
            pl.BlockSpec((tm, dim), lambda i: (i, 0)),
            pl.BlockSpec((dim_out, dim), lambda i: (0, 0)),
            pl.BlockSpec((1, dim_out), lambda i: (0, 0)),
        ],
        out_specs=pl.BlockSpec((tm, dim_out), lambda i: (i, 0)),
        out_shape=jax.ShapeDtypeStruct((b, dim_out), jnp.float32),
    )(x, q, w, bias.reshape(1, dim_out))


def kernel(x, embed_weight, proj_w, proj_b):
    b, dim = x.shape
    ind = _argmin_call(x, embed_weight, tm=8192, nk=512).reshape(b)
    quantized = _make_sc_gather(dim, b)(embed_weight, ind)
    return _proj_call(x, quantized, proj_w, proj_b, tm=1024)
